# baseline (speedup 1.0000x reference)
import functools

import jax
import jax.numpy as jnp
from jax import lax
from jax.experimental import pallas as pl
from jax.experimental.pallas import tpu as pltpu
from jax.experimental.pallas import tpu_sc as plsc

N_NODES = 10000
D_HID = 128
N_CLASSES = 40

NC = 2
NS = 16
CHUNK = 128
E_PAD = 327680
N_ACC = 10240
STRIPE = N_ACC // NS
ZROWS = 128
N_CHUNKS = E_PAD // (NS * CHUNK)

NBUF = 4
NGSEM = 3
NSEM = 2


def _zero_acc(zbuf_v, acc, s, D):
  def zbody(i, carry):
    for cb in range(D // 16):
      zbuf_v[i, pl.ds(cb * 16, 16)] = jnp.zeros((16,), jnp.float32)
    return carry
  lax.fori_loop(0, ZROWS, zbody, 0)
  for r in range(STRIPE // ZROWS):
    pltpu.sync_copy(zbuf_v, acc.at[pl.ds(s * STRIPE + r * ZROWS, ZROWS)])


def _scale_rows(rows, b, ew_v, j, D):
  for g in range(CHUNK // 16):
    ewg = ew_v[j, pl.ds(g * 16, 16)]
    for t in range(16):
      i = g * 16 + t
      wv = lax.gather(
          ewg, jnp.full((16, 1), t, jnp.int32),
          dimension_numbers=lax.GatherDimensionNumbers(
              offset_dims=(), collapsed_slice_dims=(0,),
              start_index_map=(0,)),
          slice_sizes=(1,),
          mode=lax.GatherScatterMode.PROMISE_IN_BOUNDS)
      for cb in range(D // 16):
        sl = pl.ds(cb * 16, 16)
        rows[b, i, sl] = rows[b, i, sl] * wv


IBUF = 5


def _edge_loop(y_gather, n_chunks, src_hbm, dst_hbm, ew_hbm, s,
               src_w, dst_w, ew_w, rows, gsems, ssems, isems, acc, D):
  def idx_fetch(j):
    sl = j % IBUF
    sem = isems.at[j % NSEM]
    pltpu.async_copy(src_hbm.at[s].at[j], src_w.at[sl], sem)
    pltpu.async_copy(dst_hbm.at[s].at[j], dst_w.at[sl], sem)
    pltpu.async_copy(ew_hbm.at[s].at[j], ew_w.at[sl], sem)

  def idx_wait(j):
    sl = j % IBUF
    sem = isems.at[j % NSEM]
    pltpu.make_async_copy(src_hbm.at[s].at[j], src_w.at[sl], sem).wait()
    pltpu.make_async_copy(dst_hbm.at[s].at[j], dst_w.at[sl], sem).wait()
    pltpu.make_async_copy(ew_hbm.at[s].at[j], ew_w.at[sl], sem).wait()

  def gather_issue(j):
    pltpu.async_copy(y_gather(src_w.at[j % IBUF]), rows.at[j % NBUF],
                     gsems.at[j % NGSEM])

  def gather_wait(j):
    pltpu.make_async_copy(y_gather(src_w.at[j % IBUF]), rows.at[j % NBUF],
                          gsems.at[j % NGSEM]).wait()

  def scatter_issue(j):
    pltpu.async_copy(rows.at[j % NBUF], acc.at[dst_w.at[j % IBUF]],
                     ssems.at[j % NSEM], add=True)

  def scatter_wait(j):
    pltpu.make_async_copy(rows.at[j % NBUF], acc.at[dst_w.at[j % IBUF]],
                          ssems.at[j % NSEM]).wait()

  idx_fetch(0)
  idx_fetch(1)
  idx_wait(0)
  gather_issue(0)
  idx_fetch(2)
  idx_wait(1)
  gather_issue(1)
  idx_fetch(3)
  idx_wait(2)
  gather_issue(2)
  idx_fetch(4)

  def body(j, carry):
    gather_wait(j)
    @pl.when(j >= 1)
    def _():
      scatter_wait(j - 1)
      @pl.when(j + 4 < n_chunks)
      def _():
        idx_fetch(j + 4)
    @pl.when(j + 3 < n_chunks)
    def _():
      idx_wait(j + 3)
      gather_issue(j + 3)
    _scale_rows(rows, j % NBUF, ew_w, j % IBUF, D)
    scatter_issue(j)
    return carry
  lax.fori_loop(0, n_chunks, body, 0)
  scatter_wait(n_chunks - 1)


def _make_spmm(D):
  scratch = [
      pltpu.VMEM((IBUF, CHUNK), jnp.int32),
      pltpu.VMEM((IBUF, CHUNK), jnp.int32),
      pltpu.VMEM((IBUF, CHUNK), jnp.float32),
      pltpu.VMEM((NBUF, CHUNK, D), jnp.float32),
      pltpu.VMEM((ZROWS, D), jnp.float32),
      pltpu.VMEM_SHARED((N_ACC, D), jnp.float32),
      pltpu.SemaphoreType.DMA((NGSEM,)),
      pltpu.SemaphoreType.DMA((NSEM,)),
      pltpu.SemaphoreType.DMA((NSEM,)),
  ]

  @functools.partial(
      pl.kernel,
      out_type=jax.ShapeDtypeStruct((NC, N_ACC, D), jnp.float32),
      mesh=plsc.VectorSubcoreMesh(core_axis_name="c", subcore_axis_name="s"),
      scratch_types=scratch,
      compiler_params=pltpu.CompilerParams(use_tc_tiling_on_sc=False),
  )
  def spmm(y_hbm, src_hbm, dst_hbm, ew_hbm, out_hbm,
           src_w, dst_w, ew_w, rows, zbuf_v, acc, gsems, ssems, isems):
    c = lax.axis_index("c")
    s = lax.axis_index("s")

    _zero_acc(zbuf_v, acc, s, D)

    plsc.subcore_barrier()

    _edge_loop(lambda idx: y_hbm.at[c].at[idx], N_CHUNKS,
               src_hbm, dst_hbm, ew_hbm, s,
               src_w, dst_w, ew_w, rows, gsems, ssems, isems, acc, D)

    plsc.subcore_barrier()
    pltpu.sync_copy(acc.at[pl.ds(s * STRIPE, STRIPE)],
                    out_hbm.at[c, pl.ds(s * STRIPE, STRIPE)])

  return spmm


_spmm_l1 = _make_spmm(64)
_spmm_l2 = _make_spmm(32)


def _mm1_body(x_ref, w_ref, o_ref):
  y = jnp.dot(x_ref[...], w_ref[...], preferred_element_type=jnp.float32)
  o_ref[0] = y[:, :64]
  o_ref[1] = y[:, 64:]


def _fuse_body(p_ref, b1_ref, w2_ref, o_ref):
  b1 = b1_ref[...]
  h0 = jnp.maximum(p_ref[0] + b1[None, :64], 0.0)
  h1 = jnp.maximum(p_ref[1] + b1[None, 64:], 0.0)
  w2 = w2_ref[...]
  for c in range(NC):
    o_ref[c] = (
        jnp.dot(h0, w2[:64, c * 32:(c + 1) * 32],
                preferred_element_type=jnp.float32)
        + jnp.dot(h1, w2[64:, c * 32:(c + 1) * 32],
                  preferred_element_type=jnp.float32))


def _final_body(q_ref, b2_ref, o_ref):
  o_ref[...] = (jnp.concatenate([q_ref[0], q_ref[1][:, :16]], axis=1)
                + b2_ref[...][None, :])


@jax.jit
def kernel(in_feat, edge_index, edge_weight, W1, b1, W2, b2):
  src = edge_index[0].astype(jnp.int32)
  dst = edge_index[1].astype(jnp.int32)
  ew = edge_weight.astype(jnp.float32)

  npad = E_PAD - src.shape[0]
  pad_idx = (jnp.arange(npad, dtype=jnp.int32) * 13) % N_NODES
  src = jnp.concatenate([src, pad_idx]).reshape(NS, N_CHUNKS, CHUNK)
  dst = jnp.concatenate([dst, pad_idx]).reshape(NS, N_CHUNKS, CHUNK)
  ew = jnp.concatenate([ew, jnp.zeros((npad,), jnp.float32)])
  ew = ew.reshape(NS, N_CHUNKS, CHUNK)

  y1s = pl.pallas_call(
      _mm1_body,
      out_shape=jax.ShapeDtypeStruct((NC, N_NODES, 64), jnp.float32),
  )(in_feat, W1)
  p1 = _spmm_l1(y1s, src, dst, ew)

  w2p = jnp.pad(W2, ((0, 0), (0, 24)))
  y2s = pl.pallas_call(
      _fuse_body,
      out_shape=jax.ShapeDtypeStruct((NC, N_ACC, 32), jnp.float32),
  )(p1, b1, w2p)
  p2 = _spmm_l2(y2s, src, dst, ew)

  b2p = jnp.pad(b2, (0, 8))
  outp = pl.pallas_call(
      _final_body,
      out_shape=jax.ShapeDtypeStruct((N_ACC, 48), jnp.float32),
  )(p2, b2p)
  return outp[:N_NODES, :N_CLASSES]


# speedup vs baseline: 1.0117x; 1.0117x over previous
import functools

import jax
import jax.numpy as jnp
from jax import lax
from jax.experimental import pallas as pl
from jax.experimental.pallas import tpu as pltpu
from jax.experimental.pallas import tpu_sc as plsc

N_NODES = 10000
D_HID = 128
N_CLASSES = 40

NC = 2
NS = 16
CHUNK = 128
E_PAD = 327680
N_ACC = 10240
STRIPE = N_ACC // NS
ZROWS = 128
N_CHUNKS = E_PAD // (NS * CHUNK)

NBUF = 4
NGSEM = 3
NSEM = 2


def _zero_acc(zbuf_v, acc, s, D):
  def zbody(i, carry):
    for cb in range(D // 16):
      zbuf_v[i, pl.ds(cb * 16, 16)] = jnp.zeros((16,), jnp.float32)
    return carry
  lax.fori_loop(0, ZROWS, zbody, 0)
  for r in range(STRIPE // ZROWS):
    pltpu.sync_copy(zbuf_v, acc.at[pl.ds(s * STRIPE + r * ZROWS, ZROWS)])


def _scale_rows(rows, b, ew_v, j, D):
  for g in range(CHUNK // 16):
    ewg = ew_v[j, pl.ds(g * 16, 16)]
    for t in range(16):
      i = g * 16 + t
      wv = lax.gather(
          ewg, jnp.full((16, 1), t, jnp.int32),
          dimension_numbers=lax.GatherDimensionNumbers(
              offset_dims=(), collapsed_slice_dims=(0,),
              start_index_map=(0,)),
          slice_sizes=(1,),
          mode=lax.GatherScatterMode.PROMISE_IN_BOUNDS)
      for cb in range(D // 16):
        sl = pl.ds(cb * 16, 16)
        rows[b, i, sl] = rows[b, i, sl] * wv


IBUF = 5


def _edge_loop(y_gather, n_chunks, src_hbm, dst_hbm, ew_hbm, s,
               src_w, dst_w, ew_w, rows, gsems, ssems, isems, acc, D):
  def idx_fetch(j):
    sl = j % IBUF
    sem = isems.at[j % NSEM]
    pltpu.async_copy(src_hbm.at[s].at[j], src_w.at[sl], sem)
    pltpu.async_copy(dst_hbm.at[s].at[j], dst_w.at[sl], sem)
    pltpu.async_copy(ew_hbm.at[s].at[j], ew_w.at[sl], sem)

  def idx_wait(j):
    sl = j % IBUF
    sem = isems.at[j % NSEM]
    pltpu.make_async_copy(src_hbm.at[s].at[j], src_w.at[sl], sem).wait()
    pltpu.make_async_copy(dst_hbm.at[s].at[j], dst_w.at[sl], sem).wait()
    pltpu.make_async_copy(ew_hbm.at[s].at[j], ew_w.at[sl], sem).wait()

  def gather_issue(j):
    pltpu.async_copy(y_gather(src_w.at[j % IBUF]), rows.at[j % NBUF],
                     gsems.at[j % NGSEM])

  def gather_wait(j):
    pltpu.make_async_copy(y_gather(src_w.at[j % IBUF]), rows.at[j % NBUF],
                          gsems.at[j % NGSEM]).wait()

  def scatter_issue(j):
    pltpu.async_copy(rows.at[j % NBUF], acc.at[dst_w.at[j % IBUF]],
                     ssems.at[j % NSEM], add=True)

  def scatter_wait(j):
    pltpu.make_async_copy(rows.at[j % NBUF], acc.at[dst_w.at[j % IBUF]],
                          ssems.at[j % NSEM]).wait()

  idx_fetch(0)
  idx_fetch(1)
  idx_wait(0)
  gather_issue(0)
  idx_fetch(2)
  idx_wait(1)
  gather_issue(1)
  idx_fetch(3)
  idx_wait(2)
  gather_issue(2)
  idx_fetch(4)

  def body(j, carry):
    gather_wait(j)
    @pl.when(j >= 1)
    def _():
      scatter_wait(j - 1)
      @pl.when(j + 4 < n_chunks)
      def _():
        idx_fetch(j + 4)
    @pl.when(j + 3 < n_chunks)
    def _():
      idx_wait(j + 3)
      gather_issue(j + 3)
    _scale_rows(rows, j % NBUF, ew_w, j % IBUF, D)
    scatter_issue(j)
    return carry
  lax.fori_loop(0, n_chunks, body, 0)
  scatter_wait(n_chunks - 1)


def _make_spmm(D, with_bias=False):
  scratch = [
      pltpu.VMEM((IBUF, CHUNK), jnp.int32),
      pltpu.VMEM((IBUF, CHUNK), jnp.int32),
      pltpu.VMEM((IBUF, CHUNK), jnp.float32),
      pltpu.VMEM((NBUF, CHUNK, D), jnp.float32),
      pltpu.VMEM((ZROWS, D), jnp.float32),
      pltpu.VMEM_SHARED((N_ACC, D), jnp.float32),
      pltpu.SemaphoreType.DMA((NGSEM,)),
      pltpu.SemaphoreType.DMA((NSEM,)),
      pltpu.SemaphoreType.DMA((NSEM,)),
      pltpu.VMEM((D,), jnp.float32),
  ]

  def body(y_hbm, src_hbm, dst_hbm, ew_hbm, b_hbm, out_hbm,
           src_w, dst_w, ew_w, rows, zbuf_v, acc, gsems, ssems, isems,
           bias_v):
    c = lax.axis_index("c")
    s = lax.axis_index("s")

    _zero_acc(zbuf_v, acc, s, D)

    plsc.subcore_barrier()

    _edge_loop(lambda idx: y_hbm.at[c].at[idx], N_CHUNKS,
               src_hbm, dst_hbm, ew_hbm, s,
               src_w, dst_w, ew_w, rows, gsems, ssems, isems, acc, D)

    plsc.subcore_barrier()
    if with_bias:
      pltpu.sync_copy(b_hbm.at[c], bias_v)
      for r in range(STRIPE // ZROWS):
        base = s * STRIPE + r * ZROWS
        pltpu.sync_copy(acc.at[pl.ds(base, ZROWS)], zbuf_v)
        def add_b(i, carry):
          for cb in range(D // 16):
            sl = pl.ds(cb * 16, 16)
            zbuf_v[i, sl] = zbuf_v[i, sl] + bias_v[sl]
          return carry
        lax.fori_loop(0, ZROWS, add_b, 0)
        pltpu.sync_copy(zbuf_v, out_hbm.at[c, pl.ds(base, ZROWS)])
    else:
      pltpu.sync_copy(acc.at[pl.ds(s * STRIPE, STRIPE)],
                      out_hbm.at[c, pl.ds(s * STRIPE, STRIPE)])

  kern = functools.partial(
      pl.kernel,
      out_type=jax.ShapeDtypeStruct((NC, N_ACC, D), jnp.float32),
      mesh=plsc.VectorSubcoreMesh(core_axis_name="c", subcore_axis_name="s"),
      scratch_types=scratch,
      compiler_params=pltpu.CompilerParams(use_tc_tiling_on_sc=False),
  )
  if with_bias:
    return kern(body)

  def nobias(y_hbm, src_hbm, dst_hbm, ew_hbm, out_hbm, *rest):
    return body(y_hbm, src_hbm, dst_hbm, ew_hbm, None, out_hbm, *rest)
  return kern(nobias)


_spmm_l1 = _make_spmm(64)
_spmm_l2 = _make_spmm(32, with_bias=True)


def _mm1_body(x_ref, w_ref, o_ref):
  y = jnp.dot(x_ref[...], w_ref[...], preferred_element_type=jnp.float32)
  o_ref[0] = y[:, :64]
  o_ref[1] = y[:, 64:]


def _fuse_body(p_ref, b1_ref, w2_ref, o_ref):
  b1 = b1_ref[...]
  h0 = jnp.maximum(p_ref[0] + b1[None, :64], 0.0)
  h1 = jnp.maximum(p_ref[1] + b1[None, 64:], 0.0)
  w2 = w2_ref[...]
  for c in range(NC):
    o_ref[c] = (
        jnp.dot(h0, w2[:64, c * 32:(c + 1) * 32],
                preferred_element_type=jnp.float32)
        + jnp.dot(h1, w2[64:, c * 32:(c + 1) * 32],
                  preferred_element_type=jnp.float32))


@jax.jit
def kernel(in_feat, edge_index, edge_weight, W1, b1, W2, b2):
  src = edge_index[0].astype(jnp.int32)
  dst = edge_index[1].astype(jnp.int32)
  ew = edge_weight.astype(jnp.float32)

  npad = E_PAD - src.shape[0]
  pad_idx = (jnp.arange(npad, dtype=jnp.int32) * 13) % N_NODES
  src = jnp.concatenate([src, pad_idx]).reshape(NS, N_CHUNKS, CHUNK)
  dst = jnp.concatenate([dst, pad_idx]).reshape(NS, N_CHUNKS, CHUNK)
  ew = jnp.concatenate([ew, jnp.zeros((npad,), jnp.float32)])
  ew = ew.reshape(NS, N_CHUNKS, CHUNK)

  y1s = pl.pallas_call(
      _mm1_body,
      out_shape=jax.ShapeDtypeStruct((NC, N_NODES, 64), jnp.float32),
  )(in_feat, W1)
  p1 = _spmm_l1(y1s, src, dst, ew)

  w2p = jnp.pad(W2, ((0, 0), (0, 24)))
  y2s = pl.pallas_call(
      _fuse_body,
      out_shape=jax.ShapeDtypeStruct((NC, N_ACC, 32), jnp.float32),
  )(p1, b1, w2p)
  b2s = jnp.pad(b2, (0, 24)).reshape(NC, 32)
  p2 = _spmm_l2(y2s, src, dst, ew, b2s)

  return jnp.concatenate(
      [p2[0, :N_NODES], p2[1, :N_NODES, :N_CLASSES - 32]], axis=1)
